# baseline (speedup 1.0000x reference)
import functools

import jax
import jax.numpy as jnp
from jax import lax
from jax.experimental import pallas as pl
from jax.experimental.pallas import tpu as pltpu
from jax.experimental.pallas import tpu_sc as plsc

N = 262144
K = 4
D = 32
M = 65536

_TC_BLK = 2048
_NQ = N // 4
_QB = _NQ // _TC_BLK


def _rows_body(x0_ref, x1_ref, x2_ref, x3_ref, o_ref):
    def es(x_ref):
        e = jnp.exp(x_ref[...])
        s = (e[0] + e[1]) + (e[2] + e[3])
        return jnp.transpose(s)

    t0, t1, t2, t3 = es(x0_ref), es(x1_ref), es(x2_ref), es(x3_ref)
    o_ref[...] = -jnp.concatenate([t0, t1, t2, t3], axis=1)


def _rows_tc(theta_t):
    return pl.pallas_call(
        _rows_body,
        grid=(_QB,),
        in_specs=[
            pl.BlockSpec((4, 32, _TC_BLK), lambda i, j=j: (0, 0, j * _QB + i))
            for j in range(4)
        ],
        out_specs=pl.BlockSpec((_TC_BLK, 128), lambda i: (i, 0)),
        out_shape=jax.ShapeDtypeStruct((_NQ, 128), jnp.float32),
    )(theta_t, theta_t, theta_t, theta_t)


_SC_NC = 2
_SC_NS = 16
_BPT = M // _SC_NS
_RPT = N // _SC_NS
_BLK = 1024
_NBLK = _RPT // _BLK
_JCH = _BLK // 128
_ACH = _BPT // _BLK


def _sc_body(rows_hbm, idx_hbm, obs_hbm, out_hbm, acc, buf, idxv, pout,
             semr, semi, sems):
    c = lax.axis_index("c")
    s = lax.axis_index("s")
    colbase = c * 16
    binbase = pl.multiple_of(s * _BPT, _BPT)

    pltpu.sync_copy(
        obs_hbm.at[pl.ds(binbase, _BPT), pl.ds(colbase, 16)],
        acc.at[pl.ds(binbase, _BPT), :],
    )
    plsc.subcore_barrier()

    rowbase = pl.multiple_of(s * _RPT, _BLK)
    lane0 = (s // 4) * 32 + colbase
    prow0 = pl.multiple_of((s % 4) * _RPT, _BLK)

    def _fetch(b, sl):
        pr = pl.multiple_of(prow0 + b * _BLK, _BLK)
        rbb = rowbase + b * _BLK
        dr = pltpu.async_copy(
            rows_hbm.at[pl.ds(pr, _BLK), pl.ds(lane0, 16)],
            buf.at[sl], semr.at[sl],
        )
        dis = [
            pltpu.async_copy(
                idx_hbm.at[pl.ds(pl.multiple_of(rbb + j * 128, 128), 128)],
                idxv.at[sl, j], semi.at[sl],
            )
            for j in range(_JCH)
        ]
        return dr, dis

    pend = _fetch(0, 0)
    for b in range(_NBLK):
        sl = b % 2
        dr, dis = pend
        if b + 1 < _NBLK:
            pend = _fetch(b + 1, 1 - sl)
        dr.wait()
        for d in dis:
            d.wait()
        scats = [
            pltpu.async_copy(
                buf.at[sl, pl.ds(j * 128, 128), :],
                acc.at[idxv.at[sl, j]],
                sems,
                add=True,
            )
            for j in range(_JCH)
        ]
        for d in scats:
            d.wait()
    plsc.subcore_barrier()

    acc16 = jnp.zeros((16,), jnp.float32)
    for t in range(_ACH):
        bb = pl.multiple_of(binbase + t * _BLK, _BLK)
        pltpu.sync_copy(acc.at[pl.ds(bb, _BLK), :], buf.at[t % 2])

        def _sq(r, a16, t=t):
            v = buf[t % 2, r, :]
            return a16 + v * v

        acc16 = lax.fori_loop(0, _BLK, _sq, acc16, unroll=8)
    pout[...] = acc16
    pltpu.sync_copy(pout, out_hbm.at[c, s])


@functools.cache
def _sc_call():
    return functools.partial(
        pl.kernel,
        out_type=jax.ShapeDtypeStruct((_SC_NC, _SC_NS, 16), jnp.float32),
        mesh=plsc.VectorSubcoreMesh(
            core_axis_name="c", subcore_axis_name="s",
            num_cores=_SC_NC, num_subcores=_SC_NS,
        ),
        scratch_types=[
            pltpu.VMEM_SHARED((M, 16), jnp.float32),
            pltpu.VMEM((2, _BLK, 16), jnp.float32),
            pltpu.VMEM((2, _JCH, 128), jnp.int32),
            pltpu.VMEM((16,), jnp.float32),
            pltpu.SemaphoreType.DMA((2,)),
            pltpu.SemaphoreType.DMA((2,)),
            pltpu.SemaphoreType.DMA,
        ],
        compiler_params=pltpu.CompilerParams(use_tc_tiling_on_sc=False),
    )(_sc_body)


def kernel(theta_0, obs, idx):
    rows = _rows_tc(theta_0.transpose(1, 2, 0))
    partials = _sc_call()(rows, idx.astype(jnp.int32), obs)
    return jnp.sum(partials) / (M * D)


# speedup vs baseline: 1.0445x; 1.0445x over previous
import functools

import jax
import jax.numpy as jnp
from jax import lax
from jax.experimental import pallas as pl
from jax.experimental.pallas import tpu as pltpu
from jax.experimental.pallas import tpu_sc as plsc

N = 262144
K = 4
D = 32
M = 65536

_TC_BLK = 4096
_NQ = N // 4
_QB = _NQ // _TC_BLK


def _rows_body(x0_ref, x1_ref, x2_ref, x3_ref, o_ref):
    def es(x_ref):
        e = jnp.exp(x_ref[...])
        s = (e[0] + e[1]) + (e[2] + e[3])
        return jnp.transpose(s)

    t0, t1, t2, t3 = es(x0_ref), es(x1_ref), es(x2_ref), es(x3_ref)
    o_ref[...] = -jnp.concatenate([t0, t1, t2, t3], axis=1)


def _rows_tc(theta_t):
    return pl.pallas_call(
        _rows_body,
        grid=(_QB,),
        in_specs=[
            pl.BlockSpec((4, 32, _TC_BLK), lambda i, j=j: (0, 0, j * _QB + i))
            for j in range(4)
        ],
        out_specs=pl.BlockSpec((_TC_BLK, 128), lambda i: (i, 0)),
        out_shape=jax.ShapeDtypeStruct((_NQ, 128), jnp.float32),
    )(theta_t, theta_t, theta_t, theta_t)


_SC_NC = 2
_SC_NS = 16
_BPT = M // _SC_NS
_RPT = N // _SC_NS
_BLK = 1024
_NBLK = _RPT // _BLK
_JCH = _BLK // 128
_ACH = _BPT // _BLK


def _sc_body(rows_hbm, idx_hbm, obs_hbm, out_hbm, acc, buf, idxv, pout,
             semr, semi, sems):
    c = lax.axis_index("c")
    s = lax.axis_index("s")
    colbase = c * 16
    binbase = pl.multiple_of(s * _BPT, _BPT)

    pltpu.sync_copy(
        obs_hbm.at[pl.ds(binbase, _BPT), pl.ds(colbase, 16)],
        acc.at[pl.ds(binbase, _BPT), :],
    )
    plsc.subcore_barrier()

    rowbase = pl.multiple_of(s * _RPT, _BLK)
    lane0 = (s // 4) * 32 + colbase
    prow0 = pl.multiple_of((s % 4) * _RPT, _BLK)

    def _fetch(b, sl):
        pr = pl.multiple_of(prow0 + b * _BLK, _BLK)
        rbb = rowbase + b * _BLK
        dr = pltpu.async_copy(
            rows_hbm.at[pl.ds(pr, _BLK), pl.ds(lane0, 16)],
            buf.at[sl], semr.at[sl],
        )
        dis = [
            pltpu.async_copy(
                idx_hbm.at[pl.ds(pl.multiple_of(rbb + j * 128, 128), 128)],
                idxv.at[sl, j], semi.at[sl],
            )
            for j in range(_JCH)
        ]
        return dr, dis

    pend = _fetch(0, 0)
    for b in range(_NBLK):
        sl = b % 2
        dr, dis = pend
        if b + 1 < _NBLK:
            pend = _fetch(b + 1, 1 - sl)
        dr.wait()
        for d in dis:
            d.wait()
        scats = [
            pltpu.async_copy(
                buf.at[sl, pl.ds(j * 128, 128), :],
                acc.at[idxv.at[sl, j]],
                sems,
                add=True,
            )
            for j in range(_JCH)
        ]
        for d in scats:
            d.wait()
    plsc.subcore_barrier()

    acc16 = jnp.zeros((16,), jnp.float32)
    for t in range(_ACH):
        bb = pl.multiple_of(binbase + t * _BLK, _BLK)
        pltpu.sync_copy(acc.at[pl.ds(bb, _BLK), :], buf.at[t % 2])

        def _sq(r, a16, t=t):
            v = buf[t % 2, r, :]
            return a16 + v * v

        acc16 = lax.fori_loop(0, _BLK, _sq, acc16, unroll=8)
    pout[...] = acc16
    pltpu.sync_copy(pout, out_hbm.at[c, s])


@functools.cache
def _sc_call():
    return functools.partial(
        pl.kernel,
        out_type=jax.ShapeDtypeStruct((_SC_NC, _SC_NS, 16), jnp.float32),
        mesh=plsc.VectorSubcoreMesh(
            core_axis_name="c", subcore_axis_name="s",
            num_cores=_SC_NC, num_subcores=_SC_NS,
        ),
        scratch_types=[
            pltpu.VMEM_SHARED((M, 16), jnp.float32),
            pltpu.VMEM((2, _BLK, 16), jnp.float32),
            pltpu.VMEM((2, _JCH, 128), jnp.int32),
            pltpu.VMEM((16,), jnp.float32),
            pltpu.SemaphoreType.DMA((2,)),
            pltpu.SemaphoreType.DMA((2,)),
            pltpu.SemaphoreType.DMA,
        ],
        compiler_params=pltpu.CompilerParams(use_tc_tiling_on_sc=False),
    )(_sc_body)


def kernel(theta_0, obs, idx):
    rows = _rows_tc(theta_0.transpose(1, 2, 0))
    partials = _sc_call()(rows, idx.astype(jnp.int32), obs)
    return jnp.sum(partials) / (M * D)


# speedup vs baseline: 1.0588x; 1.0137x over previous
import functools

import jax
import jax.numpy as jnp
from jax import lax
from jax.experimental import pallas as pl
from jax.experimental.pallas import tpu as pltpu
from jax.experimental.pallas import tpu_sc as plsc

N = 262144
K = 4
D = 32
M = 65536

_TC_BLK = 8192
_NQ = N // 4
_QB = _NQ // _TC_BLK


def _rows_body(x0_ref, x1_ref, x2_ref, x3_ref, o_ref):
    def es(x_ref):
        e = jnp.exp(x_ref[...])
        s = (e[0] + e[1]) + (e[2] + e[3])
        return jnp.transpose(s)

    t0, t1, t2, t3 = es(x0_ref), es(x1_ref), es(x2_ref), es(x3_ref)
    o_ref[...] = -jnp.concatenate([t0, t1, t2, t3], axis=1)


def _rows_tc(theta_t):
    return pl.pallas_call(
        _rows_body,
        grid=(_QB,),
        in_specs=[
            pl.BlockSpec((4, 32, _TC_BLK), lambda i, j=j: (0, 0, j * _QB + i))
            for j in range(4)
        ],
        out_specs=pl.BlockSpec((_TC_BLK, 128), lambda i: (i, 0)),
        out_shape=jax.ShapeDtypeStruct((_NQ, 128), jnp.float32),
    )(theta_t, theta_t, theta_t, theta_t)


_SC_NC = 2
_SC_NS = 16
_BPT = M // _SC_NS
_RPT = N // _SC_NS
_BLK = 1024
_NBLK = _RPT // _BLK
_JCH = _BLK // 128
_ACH = _BPT // _BLK


def _sc_body(rows_hbm, idx_hbm, obs_hbm, out_hbm, acc, buf, idxv, pout,
             semr, semi, sems):
    c = lax.axis_index("c")
    s = lax.axis_index("s")
    colbase = c * 16
    binbase = pl.multiple_of(s * _BPT, _BPT)

    pltpu.sync_copy(
        obs_hbm.at[pl.ds(binbase, _BPT), pl.ds(colbase, 16)],
        acc.at[pl.ds(binbase, _BPT), :],
    )
    plsc.subcore_barrier()

    rowbase = pl.multiple_of(s * _RPT, _BLK)
    lane0 = (s // 4) * 32 + colbase
    prow0 = pl.multiple_of((s % 4) * _RPT, _BLK)

    def _fetch(b, sl):
        pr = pl.multiple_of(prow0 + b * _BLK, _BLK)
        rbb = rowbase + b * _BLK
        dr = pltpu.async_copy(
            rows_hbm.at[pl.ds(pr, _BLK), pl.ds(lane0, 16)],
            buf.at[sl], semr.at[sl],
        )
        dis = [
            pltpu.async_copy(
                idx_hbm.at[pl.ds(pl.multiple_of(rbb + j * 128, 128), 128)],
                idxv.at[sl, j], semi.at[sl],
            )
            for j in range(_JCH)
        ]
        return dr, dis

    pend = _fetch(0, 0)
    for b in range(_NBLK):
        sl = b % 2
        dr, dis = pend
        if b + 1 < _NBLK:
            pend = _fetch(b + 1, 1 - sl)
        dr.wait()
        for d in dis:
            d.wait()
        scats = [
            pltpu.async_copy(
                buf.at[sl, pl.ds(j * 128, 128), :],
                acc.at[idxv.at[sl, j]],
                sems,
                add=True,
            )
            for j in range(_JCH)
        ]
        for d in scats:
            d.wait()
    plsc.subcore_barrier()

    acc16 = jnp.zeros((16,), jnp.float32)
    for t in range(_ACH):
        bb = pl.multiple_of(binbase + t * _BLK, _BLK)
        pltpu.sync_copy(acc.at[pl.ds(bb, _BLK), :], buf.at[t % 2])

        def _sq(r, a16, t=t):
            v = buf[t % 2, r, :]
            return a16 + v * v

        acc16 = lax.fori_loop(0, _BLK, _sq, acc16, unroll=8)
    pout[...] = acc16
    pltpu.sync_copy(pout, out_hbm.at[c, s])


@functools.cache
def _sc_call():
    return functools.partial(
        pl.kernel,
        out_type=jax.ShapeDtypeStruct((_SC_NC, _SC_NS, 16), jnp.float32),
        mesh=plsc.VectorSubcoreMesh(
            core_axis_name="c", subcore_axis_name="s",
            num_cores=_SC_NC, num_subcores=_SC_NS,
        ),
        scratch_types=[
            pltpu.VMEM_SHARED((M, 16), jnp.float32),
            pltpu.VMEM((2, _BLK, 16), jnp.float32),
            pltpu.VMEM((2, _JCH, 128), jnp.int32),
            pltpu.VMEM((16,), jnp.float32),
            pltpu.SemaphoreType.DMA((2,)),
            pltpu.SemaphoreType.DMA((2,)),
            pltpu.SemaphoreType.DMA,
        ],
        compiler_params=pltpu.CompilerParams(use_tc_tiling_on_sc=False),
    )(_sc_body)


def kernel(theta_0, obs, idx):
    rows = _rows_tc(theta_0.transpose(1, 2, 0))
    partials = _sc_call()(rows, idx.astype(jnp.int32), obs)
    return jnp.sum(partials) / (M * D)
